# Initial kernel scaffold; baseline (speedup 1.0000x reference)
import jax
import jax.numpy as jnp
from jax.experimental import pallas as pl


def kernel(X):
    raise NotImplementedError("write your pallas kernel here")


# speedup vs baseline: 1.7479x; 1.7479x over previous
import jax
import jax.numpy as jnp
from jax.experimental import pallas as pl
from jax.experimental.pallas import tpu as pltpu

_EPS = 1e-12
_B = 64
_S = 1024
_D = 128
_NCORES = 2
_PER = _B // _NCORES


def _sim_kernel(x_ref, out_ref):
    j = pl.program_id(1)

    @pl.when(j == 0)
    def _():
        out_ref[...] = jnp.zeros_like(out_ref)

    x = x_ref[0]
    n2 = jnp.sum(x * x, axis=1, keepdims=True)
    xn = x / jnp.maximum(jnp.sqrt(n2), _EPS)
    g = jax.lax.dot_general(
        xn, xn, (((0,), (0,)), ((), ())), preferred_element_type=jnp.float32
    )
    m = jnp.sum(xn, axis=0, keepdims=True)
    partial = jnp.sum(g * g, axis=0, keepdims=True) - 2.0 * (m * m)
    out_ref[0, :, :] += partial


def kernel(X):
    partial = pl.pallas_call(
        _sim_kernel,
        grid=(_NCORES, _PER),
        in_specs=[pl.BlockSpec((1, _S, _D), lambda i, j: (i * _PER + j, 0, 0))],
        out_specs=pl.BlockSpec((1, 1, _D), lambda i, j: (i, 0, 0)),
        out_shape=jax.ShapeDtypeStruct((_NCORES, 1, _D), jnp.float32),
        compiler_params=pltpu.CompilerParams(
            dimension_semantics=("parallel", "arbitrary"),
        ),
    )(X)
    denom = float(_B) * float(_S) * float(_S)
    return jnp.sum(partial) / denom + 1.0


# speedup vs baseline: 3.5078x; 2.0069x over previous
import jax
import jax.numpy as jnp
from jax.experimental import pallas as pl
from jax.experimental.pallas import tpu as pltpu

_EPS = 1e-12
_B = 64
_S = 1024
_D = 128
_BPS = 4
_STEPS = _B // _BPS


def _sim_kernel(x_ref, out_ref):
    j = pl.program_id(0)

    @pl.when(j == 0)
    def _():
        out_ref[...] = jnp.zeros_like(out_ref)

    acc = jnp.zeros((1, 2 * _D), jnp.float32)
    ones = jnp.ones((_S, _D), jnp.bfloat16)
    for p in range(_BPS // 2):
        xa = x_ref[2 * p]
        xb = x_ref[2 * p + 1]
        na = jax.lax.rsqrt(jnp.maximum(
            jnp.sum(xa * xa, axis=1, keepdims=True), _EPS * _EPS))
        nb = jax.lax.rsqrt(jnp.maximum(
            jnp.sum(xb * xb, axis=1, keepdims=True), _EPS * _EPS))
        y = jnp.concatenate([xa * na, xb * nb], axis=1).astype(jnp.bfloat16)
        ycat = jnp.concatenate([y, ones], axis=1)
        g3 = jax.lax.dot_general(
            ycat, y, (((0,), (0,)), ((), ())),
            preferred_element_type=jnp.float32,
        )
        ga = g3[:_D, :_D]
        gb = g3[_D:2 * _D, _D:2 * _D]
        mrow = g3[2 * _D:2 * _D + 1, :]
        q = jnp.concatenate(
            [jnp.sum(ga * ga, axis=0, keepdims=True),
             jnp.sum(gb * gb, axis=0, keepdims=True)], axis=1)
        acc = acc + q - 2.0 * (mrow * mrow)
    out_ref[0, :, :] += acc


def kernel(X):
    partial = pl.pallas_call(
        _sim_kernel,
        grid=(_STEPS,),
        in_specs=[pl.BlockSpec(
            (_BPS, _S, _D), lambda j: (j, 0, 0))],
        out_specs=pl.BlockSpec((1, 1, 2 * _D), lambda j: (0, 0, 0)),
        out_shape=jax.ShapeDtypeStruct((1, 1, 2 * _D), jnp.float32),
        compiler_params=pltpu.CompilerParams(
            dimension_semantics=("arbitrary",),
        ),
    )(X)
    denom = float(_B) * float(_S) * float(_S)
    return jnp.sum(partial) / denom + 1.0


# speedup vs baseline: 4.0275x; 1.1482x over previous
import jax
import jax.numpy as jnp
from jax.experimental import pallas as pl
from jax.experimental.pallas import tpu as pltpu

_EPS = 1e-12
_B = 64
_S = 1024
_D = 128
_BPS = 8
_STEPS = _B // _BPS


def _sim_kernel(x_ref, out_ref):
    j = pl.program_id(0)

    @pl.when(j == 0)
    def _():
        out_ref[...] = jnp.zeros_like(out_ref)

    ones_lhs = jnp.ones((8, _S), jnp.bfloat16)
    acc = jnp.zeros((1, 2 * _D), jnp.float32)
    for p in range(_BPS // 2):
        xa = x_ref[2 * p]
        xb = x_ref[2 * p + 1]
        na = jax.lax.rsqrt(jnp.maximum(
            jnp.sum(xa * xa, axis=1, keepdims=True), _EPS * _EPS))
        nb = jax.lax.rsqrt(jnp.maximum(
            jnp.sum(xb * xb, axis=1, keepdims=True), _EPS * _EPS))
        y = jnp.concatenate([xa * na, xb * nb], axis=1).astype(jnp.bfloat16)
        g = jax.lax.dot_general(
            y, y, (((0,), (0,)), ((), ())),
            preferred_element_type=jnp.float32,
        )
        mrows = jax.lax.dot_general(
            ones_lhs, y, (((1,), (0,)), ((), ())),
            preferred_element_type=jnp.float32,
        )
        ga = g[:_D, :_D]
        gb = g[_D:, _D:]
        mrow = mrows[:1, :]
        q = jnp.concatenate(
            [jnp.sum(ga * ga, axis=0, keepdims=True),
             jnp.sum(gb * gb, axis=0, keepdims=True)], axis=1)
        acc = acc + q - 2.0 * (mrow * mrow)
    out_ref[0, :, :] += acc


def kernel(X):
    partial = pl.pallas_call(
        _sim_kernel,
        grid=(_STEPS,),
        in_specs=[pl.BlockSpec(
            (_BPS, _S, _D), lambda j: (j, 0, 0))],
        out_specs=pl.BlockSpec((1, 1, 2 * _D), lambda j: (0, 0, 0)),
        out_shape=jax.ShapeDtypeStruct((1, 1, 2 * _D), jnp.float32),
        compiler_params=pltpu.CompilerParams(
            dimension_semantics=("arbitrary",),
        ),
    )(X)
    denom = float(_B) * float(_S) * float(_S)
    return jnp.sum(partial) / denom + 1.0


# speedup vs baseline: 4.0374x; 1.0025x over previous
import jax
import jax.numpy as jnp
from jax.experimental import pallas as pl
from jax.experimental.pallas import tpu as pltpu

_EPS = 1e-12
_B = 64
_S = 1024
_D = 128
_BPS = 8
_STEPS = _B // _BPS


def _sim_kernel(x0_ref, x1_ref, out_ref):
    j = pl.program_id(0)

    @pl.when(j == 0)
    def _():
        out_ref[...] = jnp.zeros_like(out_ref)

    ones_lhs = jnp.ones((8, _S), jnp.bfloat16)
    acc = jnp.zeros((1, 2 * _D), jnp.float32)
    for p in range(_BPS // 2):
        src = x0_ref if p < _BPS // 4 else x1_ref
        q2 = p % (_BPS // 4)
        xa = src[2 * q2]
        xb = src[2 * q2 + 1]
        na = jax.lax.rsqrt(jnp.maximum(
            jnp.sum(xa * xa, axis=1, keepdims=True), _EPS * _EPS))
        nb = jax.lax.rsqrt(jnp.maximum(
            jnp.sum(xb * xb, axis=1, keepdims=True), _EPS * _EPS))
        y = jnp.concatenate([xa * na, xb * nb], axis=1).astype(jnp.bfloat16)
        g = jax.lax.dot_general(
            y, y, (((0,), (0,)), ((), ())),
            preferred_element_type=jnp.float32,
        )
        mrows = jax.lax.dot_general(
            ones_lhs, y, (((1,), (0,)), ((), ())),
            preferred_element_type=jnp.float32,
        )
        ga = g[:_D, :_D]
        gb = g[_D:, _D:]
        mrow = mrows[:1, :]
        q = jnp.concatenate(
            [jnp.sum(ga * ga, axis=0, keepdims=True),
             jnp.sum(gb * gb, axis=0, keepdims=True)], axis=1)
        acc = acc + q - 2.0 * (mrow * mrow)
    out_ref[0, :, :] += acc


def kernel(X):
    partial = pl.pallas_call(
        _sim_kernel,
        grid=(_STEPS,),
        in_specs=[
            pl.BlockSpec((_BPS // 2, _S, _D), lambda j: (2 * j, 0, 0)),
            pl.BlockSpec((_BPS // 2, _S, _D), lambda j: (2 * j + 1, 0, 0)),
        ],
        out_specs=pl.BlockSpec((1, 1, 2 * _D), lambda j: (0, 0, 0)),
        out_shape=jax.ShapeDtypeStruct((1, 1, 2 * _D), jnp.float32),
        compiler_params=pltpu.CompilerParams(
            dimension_semantics=("arbitrary",),
        ),
    )(X, X)
    denom = float(_B) * float(_S) * float(_S)
    return jnp.sum(partial) / denom + 1.0
